# baseline (speedup 1.0000x reference)
import jax
import jax.numpy as jnp
from jax.experimental import pallas as pl
from jax.experimental.pallas import tpu as pltpu

_CACHE_SIZE = 4096.0
_B = 64
_HALF = 8
_ROPE = 10000.0
_NCH = 16


def _dma_kernel(sinks_smem, sinks_col, sinks_row, pos_row, kc, out,
                gat, rot, csem, gsem, wsem):
    nb = kc.shape[0]
    ch = nb // _NCH

    copies = []
    for c in range(_NCH):
        cp = pltpu.make_async_copy(kc.at[pl.ds(c * ch, ch)],
                                   out.at[pl.ds(c * ch, ch)], csem.at[c])
        cp.start()
        copies.append(cp)

    gathers = []
    for i in range(_B):
        gp = pltpu.make_async_copy(kc.at[sinks_smem[i]], gat.at[i], gsem.at[i])
        gp.start()
        gathers.append(gp)

    match = sinks_col[...] == sinks_row[...]
    pos = pos_row[...].astype(jnp.float32)
    evict = jnp.maximum(pos - _CACHE_SIZE, 0.0).astype(jnp.int32).astype(jnp.float32)
    p_total = jnp.sum(jnp.where(match, evict, 0.0), axis=1, keepdims=True)

    j = jax.lax.broadcasted_iota(jnp.int32, (_HALF, 128), 0)
    lane = jax.lax.broadcasted_iota(jnp.int32, (_HALF, 128), 1)
    f = (j * 8 + lane % 8).astype(jnp.float32)
    inv_freq = 1.0 / (_ROPE ** (f * 2.0 / 128.0))
    angle = p_total[:, :, None] * inv_freq[None, :, :]
    cos = jnp.cos(angle)[:, None, :, :]
    sin = jnp.sin(angle)[:, None, :, :]

    for gp in gathers:
        gp.wait()
    data = gat[...]
    a = data[:, :, :_HALF, :]
    b = data[:, :, _HALF:, :]
    rot[:, :, :_HALF, :] = a * cos - b * sin
    rot[:, :, _HALF:, :] = b * cos + a * sin

    for cp in copies:
        cp.wait()
    writes = []
    for i in range(_B):
        wp = pltpu.make_async_copy(rot.at[i], out.at[sinks_smem[i]], wsem.at[i])
        wp.start()
        writes.append(wp)
    for wp in writes:
        wp.wait()


def kernel(key_cache, block_tables, context_lens, positions):
    del context_lens
    nb, h, dx, bs, x = key_cache.shape
    kc = key_cache.reshape(nb, h, dx, bs * x)
    sinks = block_tables[:, 0]
    out = pl.pallas_call(
        _dma_kernel,
        in_specs=[
            pl.BlockSpec(memory_space=pltpu.MemorySpace.SMEM),
            pl.BlockSpec(memory_space=pltpu.MemorySpace.VMEM),
            pl.BlockSpec(memory_space=pltpu.MemorySpace.VMEM),
            pl.BlockSpec(memory_space=pltpu.MemorySpace.VMEM),
            pl.BlockSpec(memory_space=pltpu.MemorySpace.HBM),
        ],
        out_specs=pl.BlockSpec(memory_space=pltpu.MemorySpace.HBM),
        out_shape=jax.ShapeDtypeStruct((nb, h, dx, bs * x), key_cache.dtype),
        scratch_shapes=[
            pltpu.VMEM((_B, h, dx, bs * x), key_cache.dtype),
            pltpu.VMEM((_B, h, dx, bs * x), key_cache.dtype),
            pltpu.SemaphoreType.DMA((_NCH,)),
            pltpu.SemaphoreType.DMA((_B,)),
            pltpu.SemaphoreType.DMA((_B,)),
        ],
    )(sinks, sinks.reshape(_B, 1), sinks.reshape(1, _B),
      positions.reshape(1, _B), kc)
    return out.reshape(nb, h, dx, bs, x)


# speedup vs baseline: 12.2073x; 12.2073x over previous
import functools

import jax
import jax.numpy as jnp
from jax import lax
from jax.experimental import pallas as pl
from jax.experimental.pallas import tpu as pltpu
from jax.experimental.pallas import tpu_sc as plsc

_CACHE_SIZE = 4096.0
_B = 64
_ROPE = 10000.0
_ROW = 16384
_HROW = 2048
_HALF = 1024
_NW = 32
_RPW = _B // _NW
_CB = 128


def _tab_kernel(sinks_col, sinks_row, pos_row, cos_ref, sin_ref):
    match = sinks_col[...] == sinks_row[...]
    pos = pos_row[...].astype(jnp.float32)
    evict = jnp.maximum(pos - _CACHE_SIZE, 0.0).astype(jnp.int32).astype(jnp.float32)
    p_total = jnp.sum(jnp.where(match, evict, 0.0), axis=1, keepdims=True)

    l = jax.lax.broadcasted_iota(jnp.int32, (_B, _HALF), 1)
    f = ((l // 128) * 8 + l % 8).astype(jnp.float32)
    inv_freq = 1.0 / (_ROPE ** (f * 2.0 / 128.0))
    angle = p_total * inv_freq
    cos_ref[...] = jnp.cos(angle)
    sin_ref[...] = jnp.sin(angle)


def _sc_rotate_body(kc, cos3, sin3, idx2, out, idxv, buf, cbuf, sbuf, sem):
    w = lax.axis_index("s") * 2 + lax.axis_index("c")
    pltpu.sync_copy(idx2.at[w], idxv)
    gather = pltpu.async_copy(kc.at[idxv], buf, sem)
    pltpu.sync_copy(cos3.at[w], cbuf)
    pltpu.sync_copy(sin3.at[w], sbuf)
    gather.wait()
    for r in range(_RPW):
        for h in range(8):
            def vbody(v, carry, r=r, h=h):
                o = h * _HROW + v * 16
                a = buf[r, pl.ds(o, 16)]
                b = buf[r, pl.ds(o + _HALF, 16)]
                c = cbuf[r, pl.ds(v * 16, 16)]
                s = sbuf[r, pl.ds(v * 16, 16)]
                buf[r, pl.ds(o, 16)] = a * c - b * s
                buf[r, pl.ds(o + _HALF, 16)] = b * c + a * s
                return carry
            lax.fori_loop(0, _HALF // 16, vbody, 0)
    pltpu.sync_copy(buf, out.at[pl.ds(w * _RPW, _RPW)])


def _copy_sub_kernel(sinks_smem, rot_ref, kc_ref, out_ref):
    base = pl.program_id(0) * _CB
    out_ref[...] = kc_ref[...]
    for i in range(_B):
        r = sinks_smem[i] - base

        @pl.when(jnp.logical_and(r >= 0, r < _CB))
        def _substitute(r=r, i=i):
            out_ref[pl.ds(r, 1), :] = rot_ref[pl.ds(i, 1), :]


def kernel(key_cache, block_tables, context_lens, positions):
    del context_lens
    nb, h, dx, bs, x = key_cache.shape
    kc = key_cache.reshape(nb, _ROW)
    sinks = block_tables[:, 0]

    cos_tab, sin_tab = pl.pallas_call(
        _tab_kernel,
        out_shape=(
            jax.ShapeDtypeStruct((_B, _HALF), jnp.float32),
            jax.ShapeDtypeStruct((_B, _HALF), jnp.float32),
        ),
    )(sinks.reshape(_B, 1), sinks.reshape(1, _B), positions.reshape(1, _B))

    sc_rotate = functools.partial(
        pl.kernel,
        mesh=plsc.VectorSubcoreMesh(core_axis_name="c", subcore_axis_name="s"),
        out_type=jax.ShapeDtypeStruct((_B, _ROW), jnp.float32),
        scratch_types=[
            pltpu.VMEM((_RPW,), jnp.int32),
            pltpu.VMEM((_RPW, _ROW), jnp.float32),
            pltpu.VMEM((_RPW, _HALF), jnp.float32),
            pltpu.VMEM((_RPW, _HALF), jnp.float32),
            pltpu.SemaphoreType.DMA,
        ],
    )(_sc_rotate_body)
    rot = sc_rotate(kc, cos_tab.reshape(_NW, _RPW, _HALF),
                    sin_tab.reshape(_NW, _RPW, _HALF), sinks.reshape(_NW, _RPW))

    out = pl.pallas_call(
        _copy_sub_kernel,
        grid=(nb // _CB,),
        in_specs=[
            pl.BlockSpec(memory_space=pltpu.MemorySpace.SMEM),
            pl.BlockSpec((_B, _ROW), lambda i: (0, 0)),
            pl.BlockSpec((_CB, _ROW), lambda i: (i, 0)),
        ],
        out_specs=pl.BlockSpec((_CB, _ROW), lambda i: (i, 0)),
        out_shape=jax.ShapeDtypeStruct((nb, _ROW), key_cache.dtype),
        compiler_params=pltpu.CompilerParams(
            dimension_semantics=("arbitrary",),
        ),
    )(sinks, rot, kc)
    return out.reshape(nb, h, dx, bs, x)
